# baseline (speedup 1.0000x reference)
import functools

import jax
import jax.numpy as jnp
from jax.experimental import pallas as pl
from jax.experimental.pallas import tpu as pltpu


def _conv_pipeline_kernel(x_ref, w1_ref, w2_ref, w3_ref, w4_ref,
                          b1_ref, b2_ref, b3_ref, b4_ref,
                          sr1_ref, sr2_ref, sc1_ref, sc2_ref,
                          o_ref,
                          pad1, pad2, pad3, pad4,
                          *, R, H, W, Cin, C):
    f32 = jnp.float32
    bf16 = jnp.bfloat16
    H2, W2 = H // 2, W // 2
    H4 = H // 4
    S1 = H + 2
    S2 = H2 + 2
    M1 = R * S1 - 2
    M2 = R * S2 - 2

    def band_dot(pad_ref, w_ref, b_ref, m):
        lhs = jnp.concatenate(
            [pad_ref[0:m], pad_ref[1:1 + m], pad_ref[2:2 + m]], axis=1)
        acc = jnp.dot(lhs, w_ref[...], preferred_element_type=f32)
        return jnp.maximum(acc + b_ref[...], 0.0)

    def pool(act, sr_ref, sc_ref):
        m, n = act.shape
        ab = act.astype(bf16)
        e = jnp.maximum(
            jnp.dot(sr_ref[0], ab, preferred_element_type=f32),
            jnp.dot(sr_ref[1], ab, preferred_element_type=f32))
        z = jnp.zeros((m // 2, C), f32)
        pm = jnp.maximum(e, jnp.concatenate([e[:, C:], z], axis=1))
        return jnp.dot(pm.astype(bf16), sc_ref[0],
                       preferred_element_type=f32)

    zrow1 = jnp.zeros((1, W * Cin), bf16)
    zrow2 = jnp.zeros((1, W * C), bf16)
    zrow3 = jnp.zeros((1, W2 * C), bf16)

    for r in range(R):
        pad1[r * S1:r * S1 + 1, :] = zrow1
        pad1[r * S1 + S1 - 1:r * S1 + S1, :] = zrow1
        pad1[r * S1 + 1:r * S1 + 1 + H, :] = x_ref[r]

    act1 = band_dot(pad1, w1_ref, b1_ref, M1)
    pad2[1:1 + M1, :] = act1.astype(bf16)
    for r in range(R):
        pad2[r * S1:r * S1 + 1, :] = zrow2
        pad2[r * S1 + S1 - 1:r * S1 + S1, :] = zrow2
    act2 = band_dot(pad2, w2_ref, b2_ref, M1)
    p1 = pool(act2, sr1_ref, sc1_ref).astype(bf16)

    for r in range(R):
        pad3[r * S2:r * S2 + 1, :] = zrow3
        pad3[r * S2 + S2 - 1:r * S2 + S2, :] = zrow3
        pad3[r * S2 + 1:r * S2 + 1 + H2, :] = \
            p1[r * (S1 // 2):r * (S1 // 2) + H2]
    act3 = band_dot(pad3, w3_ref, b3_ref, M2)
    pad4[1:1 + M2, :] = act3.astype(bf16)
    for r in range(R):
        pad4[r * S2:r * S2 + 1, :] = zrow3
        pad4[r * S2 + S2 - 1:r * S2 + S2, :] = zrow3
    act4 = band_dot(pad4, w4_ref, b4_ref, M2)
    p2 = pool(act4, sr2_ref, sc2_ref).astype(bf16)

    for r in range(R):
        o_ref[r] = p2[r * (S2 // 2):r * (S2 // 2) + H4]


def _fc_kernel(p_ref, w_ref, b_ref, o_ref):
    o_ref[...] = jnp.dot(p_ref[...], w_ref[...],
                         preferred_element_type=jnp.float32) + b_ref[...]


def kernel(x, band1, band2, band3, band4, b1r, b2r, b3r, b4r,
           selr1, selc1, selr2, selc2, wf3, bfr):
    del selr1, selr2
    B, Cin, H, W = x.shape
    C = band2.shape[2] // W
    OUT = bfr.shape[-1]
    H2, W2 = H // 2, W // 2
    H4, W4 = H // 4, W // 4

    R = 16
    while B % R:
        R //= 2
    G = B // R
    S1, S2 = H + 2, H2 + 2
    M1, M2 = R * S1 - 2, R * S2 - 2

    def row_sel(m):
        i = jnp.arange(m // 2)[:, None]
        j = jnp.arange(m)[None, :]
        return jnp.stack([(j == 2 * i).astype(jnp.bfloat16),
                          (j == 2 * i + 1).astype(jnp.bfloat16)])

    sr1 = row_sel(M1)
    sr2 = row_sel(M2)

    xb = jnp.transpose(x.astype(jnp.bfloat16), (0, 2, 3, 1)).reshape(
        B, H, W * Cin)
    w1 = band1.reshape(3 * W * Cin, W * C)
    w2 = band2.reshape(3 * W * C, W * C)
    w3 = band3.reshape(3 * W2 * C, W2 * C)
    w4 = band4.reshape(3 * W2 * C, W2 * C)

    def full(a):
        if a.ndim == 3:
            return pl.BlockSpec(a.shape, lambda b: (0, 0, 0))
        return pl.BlockSpec(a.shape, lambda b: (0, 0))

    params = [w1, w2, w3, w4, b1r, b2r, b3r, b4r, sr1, sr2, selc1, selc2]

    pooled = pl.pallas_call(
        functools.partial(_conv_pipeline_kernel,
                          R=R, H=H, W=W, Cin=Cin, C=C),
        out_shape=jax.ShapeDtypeStruct((B, H4, W4 * C), jnp.bfloat16),
        grid=(G,),
        in_specs=([pl.BlockSpec((R, H, W * Cin), lambda b: (b, 0, 0))]
                  + [full(a) for a in params]),
        out_specs=pl.BlockSpec((R, H4, W4 * C), lambda b: (b, 0, 0)),
        scratch_shapes=[
            pltpu.VMEM((R * S1, W * Cin), jnp.bfloat16),
            pltpu.VMEM((R * S1, W * C), jnp.bfloat16),
            pltpu.VMEM((R * S2, W2 * C), jnp.bfloat16),
            pltpu.VMEM((R * S2, W2 * C), jnp.bfloat16),
        ],
        compiler_params=pltpu.CompilerParams(
            dimension_semantics=("arbitrary",)),
    )(xb, *params)

    feats = pooled.reshape(B, H4 * W4 * C)
    wf = wf3.reshape(H4 * W4 * C, OUT)
    GB = 2 if B % 2 == 0 else 1
    BB = B // GB
    logits = pl.pallas_call(
        _fc_kernel,
        out_shape=jax.ShapeDtypeStruct((B, OUT), jnp.float32),
        grid=(GB,),
        in_specs=[pl.BlockSpec((BB, H4 * W4 * C), lambda b: (b, 0)),
                  pl.BlockSpec(wf.shape, lambda b: (0, 0)),
                  pl.BlockSpec(bfr.shape, lambda b: (0, 0))],
        out_specs=pl.BlockSpec((BB, OUT), lambda b: (b, 0)),
        compiler_params=pltpu.CompilerParams(
            dimension_semantics=("arbitrary",)),
    )(feats, wf, bfr)
    return logits


# speedup vs baseline: 1.0241x; 1.0241x over previous
import functools

import jax
import jax.numpy as jnp
from jax.experimental import pallas as pl
from jax.experimental.pallas import tpu as pltpu


def _conv_pipeline_kernel(x_ref, w1_ref, w2_ref, w3_ref, w4_ref,
                          b1_ref, b2_ref, b3_ref, b4_ref,
                          sr1_ref, sr2_ref, sc1_ref, sc2_ref,
                          o_ref,
                          pad1, pad2, pad3, pad4,
                          *, R, H, W, Cin, C):
    f32 = jnp.float32
    bf16 = jnp.bfloat16
    H2, W2 = H // 2, W // 2
    H4 = H // 4
    S1 = H + 2
    S2 = H2 + 2
    M1 = R * S1 - 2
    M2 = R * S2 - 2

    def band_dot(pad_ref, w_ref, b_ref, m):
        lhs = jnp.concatenate(
            [pad_ref[0:m], pad_ref[1:1 + m], pad_ref[2:2 + m]], axis=1)
        w3, wk, wn = w_ref.shape
        acc = jnp.dot(lhs, w_ref[...].reshape(w3 * wk, wn),
                      preferred_element_type=f32)
        return jnp.maximum(acc + b_ref[...], 0.0)

    def pool(act, sr_ref, sc_ref):
        m, n = act.shape
        ab = act.astype(bf16)
        e = jnp.maximum(
            jnp.dot(sr_ref[0], ab, preferred_element_type=f32),
            jnp.dot(sr_ref[1], ab, preferred_element_type=f32))
        z = jnp.zeros((m // 2, C), f32)
        pm = jnp.maximum(e, jnp.concatenate([e[:, C:], z], axis=1))
        return jnp.dot(pm.astype(bf16), sc_ref[0],
                       preferred_element_type=f32)

    zrow1 = jnp.zeros((1, W * Cin), bf16)
    zrow2 = jnp.zeros((1, W * C), bf16)
    zrow3 = jnp.zeros((1, W2 * C), bf16)

    for r in range(R):
        pad1[r * S1:r * S1 + 1, :] = zrow1
        pad1[r * S1 + S1 - 1:r * S1 + S1, :] = zrow1
        pad1[r * S1 + 1:r * S1 + 1 + H, :] = x_ref[r]

    act1 = band_dot(pad1, w1_ref, b1_ref, M1)
    pad2[1:1 + M1, :] = act1.astype(bf16)
    for r in range(R):
        pad2[r * S1:r * S1 + 1, :] = zrow2
        pad2[r * S1 + S1 - 1:r * S1 + S1, :] = zrow2
    act2 = band_dot(pad2, w2_ref, b2_ref, M1)
    p1 = pool(act2, sr1_ref, sc1_ref).astype(bf16)

    for r in range(R):
        pad3[r * S2:r * S2 + 1, :] = zrow3
        pad3[r * S2 + S2 - 1:r * S2 + S2, :] = zrow3
        pad3[r * S2 + 1:r * S2 + 1 + H2, :] = \
            p1[r * (S1 // 2):r * (S1 // 2) + H2]
    act3 = band_dot(pad3, w3_ref, b3_ref, M2)
    pad4[1:1 + M2, :] = act3.astype(bf16)
    for r in range(R):
        pad4[r * S2:r * S2 + 1, :] = zrow3
        pad4[r * S2 + S2 - 1:r * S2 + S2, :] = zrow3
    act4 = band_dot(pad4, w4_ref, b4_ref, M2)
    p2 = pool(act4, sr2_ref, sc2_ref).astype(bf16)

    for r in range(R):
        o_ref[r] = p2[r * (S2 // 2):r * (S2 // 2) + H4]


def _fc_kernel(p_ref, w_ref, b_ref, o_ref):
    bb = p_ref.shape[0]
    h4 = w_ref.shape[0]
    out = w_ref.shape[2]
    acc = jnp.zeros((bb, out), jnp.float32)
    for y in range(h4):
        acc = acc + jnp.dot(p_ref[:, y, :], w_ref[y],
                            preferred_element_type=jnp.float32)
    o_ref[...] = acc + b_ref[...]


def kernel(x, band1, band2, band3, band4, b1r, b2r, b3r, b4r,
           selr1, selc1, selr2, selc2, wf3, bfr):
    del selr1, selr2
    B, Cin, H, W = x.shape
    C = band2.shape[2] // W
    OUT = bfr.shape[-1]
    H2, W2 = H // 2, W // 2
    H4, W4 = H // 4, W // 4

    R = 8
    while B % R:
        R //= 2
    G = B // R
    S1, S2 = H + 2, H2 + 2
    M1, M2 = R * S1 - 2, R * S2 - 2

    def row_sel(m):
        i = jnp.arange(m // 2)[:, None]
        j = jnp.arange(m)[None, :]
        return jnp.stack([(j == 2 * i).astype(jnp.bfloat16),
                          (j == 2 * i + 1).astype(jnp.bfloat16)])

    sr1 = row_sel(M1)
    sr2 = row_sel(M2)

    xb = jnp.transpose(x.astype(jnp.bfloat16), (0, 2, 3, 1)).reshape(
        B, H, W * Cin)
    w1, w2, w3, w4 = band1, band2, band3, band4

    def full(a):
        if a.ndim == 3:
            return pl.BlockSpec(a.shape, lambda b: (0, 0, 0))
        return pl.BlockSpec(a.shape, lambda b: (0, 0))

    params = [w1, w2, w3, w4, b1r, b2r, b3r, b4r, sr1, sr2, selc1, selc2]

    pooled = pl.pallas_call(
        functools.partial(_conv_pipeline_kernel,
                          R=R, H=H, W=W, Cin=Cin, C=C),
        out_shape=jax.ShapeDtypeStruct((B, H4, W4 * C), jnp.bfloat16),
        grid=(G,),
        in_specs=([pl.BlockSpec((R, H, W * Cin), lambda b: (b, 0, 0))]
                  + [full(a) for a in params]),
        out_specs=pl.BlockSpec((R, H4, W4 * C), lambda b: (b, 0, 0)),
        scratch_shapes=[
            pltpu.VMEM((R * S1, W * Cin), jnp.bfloat16),
            pltpu.VMEM((R * S1, W * C), jnp.bfloat16),
            pltpu.VMEM((R * S2, W2 * C), jnp.bfloat16),
            pltpu.VMEM((R * S2, W2 * C), jnp.bfloat16),
        ],
        compiler_params=pltpu.CompilerParams(
            dimension_semantics=("arbitrary",)),
    )(xb, *params)

    GB = 2 if B % 2 == 0 else 1
    BB = B // GB
    logits = pl.pallas_call(
        _fc_kernel,
        out_shape=jax.ShapeDtypeStruct((B, OUT), jnp.float32),
        grid=(GB,),
        in_specs=[pl.BlockSpec((BB, H4, W4 * C), lambda b: (b, 0, 0)),
                  pl.BlockSpec(wf3.shape, lambda b: (0, 0, 0)),
                  pl.BlockSpec(bfr.shape, lambda b: (0, 0))],
        out_specs=pl.BlockSpec((BB, OUT), lambda b: (b, 0)),
        compiler_params=pltpu.CompilerParams(
            dimension_semantics=("arbitrary",)),
    )(pooled, wf3, bfr)
    return logits
